# baseline (speedup 1.0000x reference)
import functools
import jax
import jax.numpy as jnp
from jax import lax
from jax.experimental import pallas as pl
from jax.experimental.pallas import tpu as pltpu
from jax.experimental.pallas import tpu_sc as plsc

VOCAB = 100001
POS = 200
BATCH = 16384

NC = 2
NS = 16
NW = NC * NS

TOTAL = BATCH * POS
CHUNK = 12800
IDX_SLICES = CHUNK // 128
NCHUNK = TOTAL // CHUNK // NW
PERIOD = 400
NPATCH = CHUNK // POS
LAST_OFF = (POS - 1) * VOCAB


def _sc_kernel(x0_hbm, tab_hbm, out_hbm,
               xb0, xb1, ib0, ib1, gb0, gb1, offp,
               sem_in0, sem_in1, sem_g0, sem_g1, sem_out0, sem_out1):
    wid = lax.axis_index("s") * NC + lax.axis_index("c")
    elem0 = wid * (NCHUNK * CHUNK)
    lane = lax.iota(jnp.int32, 16)

    for j in range(PERIOD // 16):
        offp[pl.ds(j * 16, 16)] = lax.rem(j * 16 + lane, POS) * VOCAB

    def in_copy(c, xb, sem):
        pltpu.async_copy(
            x0_hbm.at[pl.ds(elem0 + c * CHUNK, CHUNK)], xb, sem)

    def wait_bytes(buf, sem):
        pltpu.make_async_copy(x0_hbm.at[pl.ds(0, CHUNK)], buf, sem).wait()

    def compute_idx(xb, ib):
        def body(q, carry):
            s0 = q * PERIOD
            for u in range(PERIOD // 16):
                s = s0 + u * 16
                ib[pl.ds(s, 16)] = xb[pl.ds(s, 16)] + offp[pl.ds(u * 16, 16)]
            return carry

        lax.fori_loop(0, CHUNK // PERIOD, body, 0, unroll=False)

    def fire(ib, gb, sem):
        pltpu.async_copy(tab_hbm.at[ib], gb, sem)

    def patch(ib, gb):
        for k in range(NPATCH):
            pos = POS - 1 + POS * k
            a = pos & ~15
            m = lane == (pos - a)
            iv = ib[pl.ds(a, 16)] - LAST_OFF
            gv = gb[pl.ds(a, 16)]
            gb[pl.ds(a, 16)] = jnp.where(m, iv, gv)

    def out_copy(c, gb, sem):
        pltpu.async_copy(
            gb, out_hbm.at[pl.ds(elem0 + c * CHUNK, CHUNK)], sem)

    xbs = (xb0, xb1)
    ibs = (ib0, ib1)
    gbs = (gb0, gb1)
    sin = (sem_in0, sem_in1)
    sg = (sem_g0, sem_g1)
    sout = (sem_out0, sem_out1)

    in_copy(0, xb0, sem_in0)
    in_copy(1, xb1, sem_in1)
    wait_bytes(xb0, sem_in0)
    compute_idx(xb0, ib0)
    in_copy(2, xb0, sem_in0)

    for c in range(NCHUNK):
        b = c & 1
        if c >= 2:
            wait_bytes(gbs[b], sout[b])
        fire(ibs[b], gbs[b], sg[b])
        if c + 1 < NCHUNK:
            nb = (c + 1) & 1
            wait_bytes(xbs[nb], sin[nb])
            compute_idx(xbs[nb], ibs[nb])
            if c + 3 < NCHUNK:
                in_copy(c + 3, xbs[nb], sin[nb])
        wait_bytes(gbs[b], sg[b])
        patch(ibs[b], gbs[b])
        out_copy(c, gbs[b], sout[b])

    wait_bytes(gbs[(NCHUNK - 2) & 1], sout[(NCHUNK - 2) & 1])
    wait_bytes(gbs[(NCHUNK - 1) & 1], sout[(NCHUNK - 1) & 1])


@jax.jit
def _scramble(x0v, tab):
    mesh = plsc.VectorSubcoreMesh(core_axis_name="c", subcore_axis_name="s")
    f = functools.partial(
        pl.kernel,
        out_type=jax.ShapeDtypeStruct(x0v.shape, jnp.int32),
        mesh=mesh,
        scratch_types=[
            pltpu.VMEM((CHUNK,), jnp.int32),
            pltpu.VMEM((CHUNK,), jnp.int32),
            pltpu.VMEM((CHUNK,), jnp.int32),
            pltpu.VMEM((CHUNK,), jnp.int32),
            pltpu.VMEM((CHUNK,), jnp.int32),
            pltpu.VMEM((CHUNK,), jnp.int32),
            pltpu.VMEM((PERIOD,), jnp.int32),
            pltpu.SemaphoreType.DMA,
            pltpu.SemaphoreType.DMA,
            pltpu.SemaphoreType.DMA,
            pltpu.SemaphoreType.DMA,
            pltpu.SemaphoreType.DMA,
            pltpu.SemaphoreType.DMA,
        ],
    )(_sc_kernel)
    return f(x0v, tab)


def kernel(x0, x1, x2, scrambler):
    b, p = x0.shape
    x0v = x0.reshape(-1)
    tab = scrambler.reshape(-1)
    outv = _scramble(x0v, tab)
    return (outv.reshape(b, p), x1, x2)


# speedup vs baseline: 3.7369x; 3.7369x over previous
import functools
import jax
import jax.numpy as jnp
from jax import lax
from jax.experimental import pallas as pl
from jax.experimental.pallas import tpu as pltpu
from jax.experimental.pallas import tpu_sc as plsc

VOCAB = 100001
POS = 200
BATCH = 16384

NC = 2
NS = 16

GROUPS = POS // 8
ROW_PAD = 100352
IMG_SIZE = POS * ROW_PAD
BPT = BATCH // NS
TAIL_COL0 = 99328
TAIL_W = 1024
WA = 6144
WA0 = 3072
WA1 = 3072
WB0 = 4096
WB1 = 3072


def _mult(x, n):
    return pl.multiple_of(x, n)


def _sc_kernel(x0t, tab, tail, outt, img,
               fshA0, fshA1, fshB0, fshB1, tsh, osh, xb, ib, gb,
               sem_fi, sem_fo, sem_x, sem_g, sem_o):
    c = lax.axis_index("c")
    s = lax.axis_index("s")
    col0 = _mult(s * WA, 1024)
    bcol = _mult(s * BPT, 128)
    sa8 = s * 8
    sb8 = (s - 15) * 8

    @pl.when(s == NS - 1)
    def _():
        pltpu.sync_copy(tail, tsh)

    def flat_in(g, fa, fb):
        @pl.when(s <= 14)
        def _():
            src = tab.at[pl.ds(_mult(g * 8, 8), 8), pl.ds(col0, WA0)]
            pltpu.async_copy(src, fa.at[pl.ds(sa8, 8)], sem_fi)

        @pl.when(s == 15)
        def _():
            src = tab.at[pl.ds(_mult(g * 8, 8), 8), pl.ds(col0, WB0)]
            pltpu.async_copy(src, fb.at[pl.ds(sb8, 8)], sem_fi)

    def _rows_out(g, buf, rbase, coff, w):
        for j in range(8):
            off = _mult((g * 8 + j) * ROW_PAD + coff, 1024)
            pltpu.async_copy(buf.at[rbase + j], img.at[pl.ds(off, w)],
                             sem_fo)

    def _rows_drain(buf, rbase, w):
        for j in range(8):
            pltpu.make_async_copy(
                buf.at[rbase + j], img.at[pl.ds(0, w)], sem_fo).wait()

    def flat_out(g, fa, fb):
        @pl.when(s <= 14)
        def _():
            pltpu.make_async_copy(
                tab.at[pl.ds(0, 8), pl.ds(0, WA0)], fa.at[pl.ds(sa8, 8)],
                sem_fi).wait()
            _rows_out(g, fa, sa8, col0, WA0)
            _rows_drain(fa, sa8, WA0)
            src = tab.at[pl.ds(_mult(g * 8, 8), 8),
                         pl.ds(col0 + WA0, WA1)]
            pltpu.async_copy(src, fa.at[pl.ds(sa8, 8), pl.ds(0, WA1)],
                             sem_fi)
            pltpu.make_async_copy(
                tab.at[pl.ds(0, 8), pl.ds(0, WA1)],
                fa.at[pl.ds(sa8, 8), pl.ds(0, WA1)], sem_fi).wait()
            for j in range(8):
                off = _mult((g * 8 + j) * ROW_PAD + col0 + WA0, 1024)
                pltpu.async_copy(
                    fa.at[sa8 + j, pl.ds(0, WA1)],
                    img.at[pl.ds(off, WA1)], sem_fo)

        @pl.when(s == 15)
        def _():
            pltpu.make_async_copy(
                tab.at[pl.ds(0, 8), pl.ds(0, WB0)], fb.at[pl.ds(sb8, 8)],
                sem_fi).wait()
            _rows_out(g, fb, sb8, col0, WB0)
            _rows_drain(fb, sb8, WB0)
            src = tab.at[pl.ds(_mult(g * 8, 8), 8),
                         pl.ds(col0 + WB0, WB1)]
            pltpu.async_copy(src, fb.at[pl.ds(sb8, 8), pl.ds(0, WB1)],
                             sem_fi)
            pltpu.make_async_copy(
                tab.at[pl.ds(0, 8), pl.ds(0, WB1)],
                fb.at[pl.ds(sb8, 8), pl.ds(0, WB1)], sem_fi).wait()
            for j in range(8):
                off = _mult((g * 8 + j) * ROW_PAD + col0 + WB0, 1024)
                pltpu.async_copy(
                    fb.at[sb8 + j, pl.ds(0, WB1)],
                    img.at[pl.ds(off, WB1)], sem_fo)
            for j in range(8):
                off = _mult((g * 8 + j) * ROW_PAD + TAIL_COL0, 1024)
                pltpu.async_copy(
                    tsh.at[g * 8 + j], img.at[pl.ds(off, TAIL_W)], sem_fo)

    def flat_drain(fa, fb):
        @pl.when(s <= 14)
        def _():
            _rows_drain(fa, sa8, WA1)

        @pl.when(s == 15)
        def _():
            _rows_drain(fb, sb8, WB1)
            for j in range(8):
                pltpu.make_async_copy(
                    tsh.at[j], img.at[pl.ds(0, TAIL_W)], sem_fo).wait()

    def gather_group(g, is_last_group):
        src = x0t.at[pl.ds(_mult(g * 8, 8), 8), pl.ds(bcol, BPT)]
        pltpu.async_copy(src, xb, sem_x)
        pltpu.make_async_copy(src, xb, sem_x).wait()

        for j in range(8):
            off = (g * 8 + j) * ROW_PAD

            def body(u, carry, j=j, off=off):
                ib[pl.ds(j * BPT + u * 16, 16)] = (
                    xb[j, pl.ds(u * 16, 16)] + off)
                return carry

            lax.fori_loop(0, BPT // 16, body, 0, unroll=False)

        for j in range(8):
            pltpu.async_copy(
                img.at[ib.at[pl.ds(j * BPT, BPT)]],
                gb.at[pl.ds(j * BPT, BPT)], sem_g)
        pltpu.make_async_copy(
            img.at[pl.ds(0, 8 * BPT)], gb, sem_g).wait()

        if is_last_group:
            def pbody(u, carry):
                gb[pl.ds(7 * BPT + u * 16, 16)] = xb[7, pl.ds(u * 16, 16)]
                return carry

            lax.fori_loop(0, BPT // 16, pbody, 0, unroll=False)

        for j in range(8):
            pltpu.async_copy(
                gb.at[pl.ds(j * BPT, BPT)], osh.at[sa8 + j], sem_o)
        for j in range(8):
            pltpu.make_async_copy(
                gb.at[pl.ds(j * BPT, BPT)], osh.at[sa8 + j], sem_o).wait()
        pltpu.async_copy(
            osh.at[pl.ds(sa8, 8)],
            outt.at[pl.ds(_mult(g * 8, 8), 8), pl.ds(bcol, BPT)], sem_o)

    def drain_out():
        pltpu.make_async_copy(
            osh.at[pl.ds(sa8, 8)],
            outt.at[pl.ds(0, 8), pl.ds(0, BPT)], sem_o).wait()

    fshs = ((fshA0, fshB0), (fshA1, fshB1))

    flat_in(c, *fshs[0])
    for i in range(12):
        g = c + 2 * i
        ph = i & 1
        flat_out(g, *fshs[ph])
        flat_drain(*fshs[ph])
        plsc.subcore_barrier()
        nxt = c + 2 * (i + 1)
        if i + 1 < 12:
            flat_in(nxt, *fshs[1 - ph])
        else:
            @pl.when(c == 0)
            def _(nxt=nxt, ph=ph):
                flat_in(nxt, *fshs[1 - ph])
        if i > 0:
            drain_out()
        gather_group(g, is_last_group=False)

    @pl.when(c == 0)
    def _():
        flat_out(c + 24, *fshs[0])
        flat_drain(*fshs[0])

    plsc.subcore_barrier()

    @pl.when(c == 0)
    def _():
        drain_out()
        gather_group(c + 24, is_last_group=True)

    drain_out()


@jax.jit
def _scramble(x0t, tab, tail):
    mesh = plsc.VectorSubcoreMesh(core_axis_name="c", subcore_axis_name="s")
    f = functools.partial(
        pl.kernel,
        out_type=(
            jax.ShapeDtypeStruct((POS, BATCH), jnp.int32),
            jax.ShapeDtypeStruct((IMG_SIZE,), jnp.int32),
        ),
        mesh=mesh,
        scratch_types=[
            pltpu.VMEM_SHARED((15 * 8, WA0), jnp.int32),
            pltpu.VMEM_SHARED((15 * 8, WA0), jnp.int32),
            pltpu.VMEM_SHARED((8, WB0), jnp.int32),
            pltpu.VMEM_SHARED((8, WB0), jnp.int32),
            pltpu.VMEM_SHARED((POS, TAIL_W), jnp.int32),
            pltpu.VMEM_SHARED((NS * 8, BPT), jnp.int32),
            pltpu.VMEM((8, BPT), jnp.int32),
            pltpu.VMEM((8 * BPT,), jnp.int32),
            pltpu.VMEM((8 * BPT,), jnp.int32),
            pltpu.SemaphoreType.DMA,
            pltpu.SemaphoreType.DMA,
            pltpu.SemaphoreType.DMA,
            pltpu.SemaphoreType.DMA,
            pltpu.SemaphoreType.DMA,
        ],
    )(_sc_kernel)
    return f(x0t, tab, tail)


def kernel(x0, x1, x2, scrambler):
    tail = jnp.pad(scrambler[:, TAIL_COL0:],
                   ((0, 0), (0, TAIL_W - (VOCAB - TAIL_COL0))))
    outt, _ = _scramble(x0.T, scrambler, tail)
    return (outt.T, x1, x2)


# speedup vs baseline: 4.4958x; 1.2031x over previous
import functools
import jax
import jax.numpy as jnp
from jax import lax
from jax.experimental import pallas as pl
from jax.experimental.pallas import tpu as pltpu
from jax.experimental.pallas import tpu_sc as plsc

VOCAB = 100001
POS = 200
BATCH = 16384

NC = 2
NS = 16

GROUPS = POS // 8
ROW_PAD = 100352
IMG_SIZE = POS * ROW_PAD
BPT = BATCH // NS
TAIL_COL0 = 99328
TAIL_W = 1024
WA = 6144
WA0 = 3072
WA1 = 3072
WB0 = 4096
WB1 = 3072


def _mult(x, n):
    return pl.multiple_of(x, n)


def _sc_kernel(x0t, tab, tail, outt, img,
               fshA0, fshA1, fshB0, fshB1, tsh, osh, xb, ib, gb,
               sem_fi, sem_fo, sem_x, sem_g, sem_o):
    c = lax.axis_index("c")
    s = lax.axis_index("s")
    col0 = _mult(s * WA, 1024)
    bcol = _mult(s * BPT, 128)
    sa8 = s * 8
    sb8 = (s - 15) * 8

    @pl.when(s == NS - 1)
    def _():
        pltpu.sync_copy(tail, tsh)

    def flat_in(g, fa, fb):
        @pl.when(s <= 14)
        def _():
            src = tab.at[pl.ds(_mult(g * 8, 8), 8), pl.ds(col0, WA0)]
            pltpu.async_copy(src, fa.at[pl.ds(sa8, 8)], sem_fi)

        @pl.when(s == 15)
        def _():
            src = tab.at[pl.ds(_mult(g * 8, 8), 8), pl.ds(col0, WB0)]
            pltpu.async_copy(src, fb.at[pl.ds(sb8, 8)], sem_fi)

    def _rows_out(g, buf, rbase, coff, w):
        for j in range(8):
            off = _mult((g * 8 + j) * ROW_PAD + coff, 1024)
            pltpu.async_copy(buf.at[rbase + j], img.at[pl.ds(off, w)],
                             sem_fo)

    def _rows_drain(buf, rbase, w):
        for j in range(8):
            pltpu.make_async_copy(
                buf.at[rbase + j], img.at[pl.ds(0, w)], sem_fo).wait()

    def flat_out(g, fa, fb):
        @pl.when(s <= 14)
        def _():
            pltpu.make_async_copy(
                tab.at[pl.ds(0, 8), pl.ds(0, WA0)], fa.at[pl.ds(sa8, 8)],
                sem_fi).wait()
            _rows_out(g, fa, sa8, col0, WA0)
            _rows_drain(fa, sa8, WA0)
            src = tab.at[pl.ds(_mult(g * 8, 8), 8),
                         pl.ds(col0 + WA0, WA1)]
            pltpu.async_copy(src, fa.at[pl.ds(sa8, 8), pl.ds(0, WA1)],
                             sem_fi)
            pltpu.make_async_copy(
                tab.at[pl.ds(0, 8), pl.ds(0, WA1)],
                fa.at[pl.ds(sa8, 8), pl.ds(0, WA1)], sem_fi).wait()
            for j in range(8):
                off = _mult((g * 8 + j) * ROW_PAD + col0 + WA0, 1024)
                pltpu.async_copy(
                    fa.at[sa8 + j, pl.ds(0, WA1)],
                    img.at[pl.ds(off, WA1)], sem_fo)

        @pl.when(s == 15)
        def _():
            pltpu.make_async_copy(
                tab.at[pl.ds(0, 8), pl.ds(0, WB0)], fb.at[pl.ds(sb8, 8)],
                sem_fi).wait()
            _rows_out(g, fb, sb8, col0, WB0)
            _rows_drain(fb, sb8, WB0)
            src = tab.at[pl.ds(_mult(g * 8, 8), 8),
                         pl.ds(col0 + WB0, WB1)]
            pltpu.async_copy(src, fb.at[pl.ds(sb8, 8), pl.ds(0, WB1)],
                             sem_fi)
            pltpu.make_async_copy(
                tab.at[pl.ds(0, 8), pl.ds(0, WB1)],
                fb.at[pl.ds(sb8, 8), pl.ds(0, WB1)], sem_fi).wait()
            for j in range(8):
                off = _mult((g * 8 + j) * ROW_PAD + col0 + WB0, 1024)
                pltpu.async_copy(
                    fb.at[sb8 + j, pl.ds(0, WB1)],
                    img.at[pl.ds(off, WB1)], sem_fo)
            for j in range(8):
                off = _mult((g * 8 + j) * ROW_PAD + TAIL_COL0, 1024)
                pltpu.async_copy(
                    tsh.at[g * 8 + j], img.at[pl.ds(off, TAIL_W)], sem_fo)

    def flat_drain(fa, fb):
        @pl.when(s <= 14)
        def _():
            _rows_drain(fa, sa8, WA1)

        @pl.when(s == 15)
        def _():
            _rows_drain(fb, sb8, WB1)
            for j in range(8):
                pltpu.make_async_copy(
                    tsh.at[j], img.at[pl.ds(0, TAIL_W)], sem_fo).wait()

    def start_gather(g):
        src = x0t.at[pl.ds(_mult(g * 8, 8), 8), pl.ds(bcol, BPT)]
        pltpu.async_copy(src, xb, sem_x)
        pltpu.make_async_copy(src, xb, sem_x).wait()

        for j in range(8):
            off = (g * 8 + j) * ROW_PAD

            def body(u, carry, j=j, off=off):
                ib[pl.ds(j * BPT + u * 16, 16)] = (
                    xb[j, pl.ds(u * 16, 16)] + off)
                return carry

            lax.fori_loop(0, BPT // 16, body, 0, unroll=False)

        for j in range(8):
            pltpu.async_copy(
                img.at[ib.at[pl.ds(j * BPT, BPT)]],
                gb.at[pl.ds(j * BPT, BPT)], sem_g)

    def finish_gather(g, is_last_group, first):
        pltpu.make_async_copy(
            img.at[pl.ds(0, 8 * BPT)], gb, sem_g).wait()

        if is_last_group:
            def pbody(u, carry):
                gb[pl.ds(7 * BPT + u * 16, 16)] = xb[7, pl.ds(u * 16, 16)]
                return carry

            lax.fori_loop(0, BPT // 16, pbody, 0, unroll=False)

        if not first:
            drain_out()
        for j in range(8):
            pltpu.async_copy(
                gb.at[pl.ds(j * BPT, BPT)], osh.at[sa8 + j], sem_o)
        for j in range(8):
            pltpu.make_async_copy(
                gb.at[pl.ds(j * BPT, BPT)], osh.at[sa8 + j], sem_o).wait()
        pltpu.async_copy(
            osh.at[pl.ds(sa8, 8)],
            outt.at[pl.ds(_mult(g * 8, 8), 8), pl.ds(bcol, BPT)], sem_o)

    def drain_out():
        pltpu.make_async_copy(
            osh.at[pl.ds(sa8, 8)],
            outt.at[pl.ds(0, 8), pl.ds(0, BPT)], sem_o).wait()

    fshs = ((fshA0, fshB0), (fshA1, fshB1))

    flat_in(c, *fshs[0])
    for i in range(12):
        g = c + 2 * i
        ph = i & 1
        flat_out(g, *fshs[ph])
        flat_drain(*fshs[ph])
        plsc.subcore_barrier()
        nxt = c + 2 * (i + 1)
        if i + 1 < 12:
            flat_in(nxt, *fshs[1 - ph])
        else:
            @pl.when(c == 0)
            def _(nxt=nxt, ph=ph):
                flat_in(nxt, *fshs[1 - ph])
        if i > 0:
            finish_gather(g - 2, is_last_group=False, first=(i == 1))
        start_gather(g)

    @pl.when(c == 0)
    def _():
        flat_out(c + 24, *fshs[0])
        flat_drain(*fshs[0])

    plsc.subcore_barrier()

    finish_gather(c + 22, is_last_group=False, first=False)

    @pl.when(c == 0)
    def _():
        start_gather(c + 24)
        finish_gather(c + 24, is_last_group=True, first=False)

    drain_out()


@jax.jit
def _scramble(x0t, tab, tail):
    mesh = plsc.VectorSubcoreMesh(core_axis_name="c", subcore_axis_name="s")
    f = functools.partial(
        pl.kernel,
        out_type=(
            jax.ShapeDtypeStruct((POS, BATCH), jnp.int32),
            jax.ShapeDtypeStruct((IMG_SIZE,), jnp.int32),
        ),
        mesh=mesh,
        scratch_types=[
            pltpu.VMEM_SHARED((15 * 8, WA0), jnp.int32),
            pltpu.VMEM_SHARED((15 * 8, WA0), jnp.int32),
            pltpu.VMEM_SHARED((8, WB0), jnp.int32),
            pltpu.VMEM_SHARED((8, WB0), jnp.int32),
            pltpu.VMEM_SHARED((POS, TAIL_W), jnp.int32),
            pltpu.VMEM_SHARED((NS * 8, BPT), jnp.int32),
            pltpu.VMEM((8, BPT), jnp.int32),
            pltpu.VMEM((8 * BPT,), jnp.int32),
            pltpu.VMEM((8 * BPT,), jnp.int32),
            pltpu.SemaphoreType.DMA,
            pltpu.SemaphoreType.DMA,
            pltpu.SemaphoreType.DMA,
            pltpu.SemaphoreType.DMA,
            pltpu.SemaphoreType.DMA,
        ],
    )(_sc_kernel)
    return f(x0t, tab, tail)


def kernel(x0, x1, x2, scrambler):
    tail = jnp.pad(scrambler[:, TAIL_COL0:],
                   ((0, 0), (0, TAIL_W - (VOCAB - TAIL_COL0))))
    outt, _ = _scramble(x0.T, scrambler, tail)
    return (outt.T, x1, x2)


# speedup vs baseline: 4.6966x; 1.0447x over previous
import functools
import jax
import jax.numpy as jnp
from jax import lax
from jax.experimental import pallas as pl
from jax.experimental.pallas import tpu as pltpu
from jax.experimental.pallas import tpu_sc as plsc

VOCAB = 100001
POS = 200
BATCH = 16384

NC = 2
NS = 16

GROUPS = POS // 8
ROW_PAD = 100352
IMG_SIZE = POS * ROW_PAD
BPT = BATCH // NS
TAIL_COL0 = 99328
TAIL_W = 1024
WA = 6144
WA0 = 3072
WA1 = 3072
WB0 = 4096
WB1 = 3072


def _mult(x, n):
    return pl.multiple_of(x, n)


def _sc_kernel(x0t, tab, tail, outt, img,
               fshA0, fshA1, fshB0, fshB1, tsh, osh, xb, xb2, ib, gb,
               sem_fi, sem_fo, sem_x, sem_x2, sem_g, sem_o):
    c = lax.axis_index("c")
    s = lax.axis_index("s")
    col0 = _mult(s * WA, 1024)
    bcol = _mult(s * BPT, 128)
    sa8 = s * 8
    sb8 = (s - 15) * 8

    @pl.when(s == NS - 1)
    def _():
        pltpu.sync_copy(tail, tsh)

    def flat_in(g, fa, fb):
        @pl.when(s <= 14)
        def _():
            src = tab.at[pl.ds(_mult(g * 8, 8), 8), pl.ds(col0, WA0)]
            pltpu.async_copy(src, fa.at[pl.ds(sa8, 8)], sem_fi)

        @pl.when(s == 15)
        def _():
            src = tab.at[pl.ds(_mult(g * 8, 8), 8), pl.ds(col0, WB0)]
            pltpu.async_copy(src, fb.at[pl.ds(sb8, 8)], sem_fi)

    def _rows_out(g, buf, rbase, coff, w):
        for j in range(8):
            off = _mult((g * 8 + j) * ROW_PAD + coff, 1024)
            pltpu.async_copy(buf.at[rbase + j], img.at[pl.ds(off, w)],
                             sem_fo)

    def _rows_drain(buf, rbase, w):
        for j in range(8):
            pltpu.make_async_copy(
                buf.at[rbase + j], img.at[pl.ds(0, w)], sem_fo).wait()

    def flat_out(g, fa, fb):
        @pl.when(s <= 14)
        def _():
            pltpu.make_async_copy(
                tab.at[pl.ds(0, 8), pl.ds(0, WA0)], fa.at[pl.ds(sa8, 8)],
                sem_fi).wait()
            _rows_out(g, fa, sa8, col0, WA0)
            _rows_drain(fa, sa8, WA0)
            src = tab.at[pl.ds(_mult(g * 8, 8), 8),
                         pl.ds(col0 + WA0, WA1)]
            pltpu.async_copy(src, fa.at[pl.ds(sa8, 8), pl.ds(0, WA1)],
                             sem_fi)
            pltpu.make_async_copy(
                tab.at[pl.ds(0, 8), pl.ds(0, WA1)],
                fa.at[pl.ds(sa8, 8), pl.ds(0, WA1)], sem_fi).wait()
            for j in range(8):
                off = _mult((g * 8 + j) * ROW_PAD + col0 + WA0, 1024)
                pltpu.async_copy(
                    fa.at[sa8 + j, pl.ds(0, WA1)],
                    img.at[pl.ds(off, WA1)], sem_fo)

        @pl.when(s == 15)
        def _():
            pltpu.make_async_copy(
                tab.at[pl.ds(0, 8), pl.ds(0, WB0)], fb.at[pl.ds(sb8, 8)],
                sem_fi).wait()
            _rows_out(g, fb, sb8, col0, WB0)
            _rows_drain(fb, sb8, WB0)
            src = tab.at[pl.ds(_mult(g * 8, 8), 8),
                         pl.ds(col0 + WB0, WB1)]
            pltpu.async_copy(src, fb.at[pl.ds(sb8, 8), pl.ds(0, WB1)],
                             sem_fi)
            pltpu.make_async_copy(
                tab.at[pl.ds(0, 8), pl.ds(0, WB1)],
                fb.at[pl.ds(sb8, 8), pl.ds(0, WB1)], sem_fi).wait()
            for j in range(8):
                off = _mult((g * 8 + j) * ROW_PAD + col0 + WB0, 1024)
                pltpu.async_copy(
                    fb.at[sb8 + j, pl.ds(0, WB1)],
                    img.at[pl.ds(off, WB1)], sem_fo)
            for j in range(8):
                off = _mult((g * 8 + j) * ROW_PAD + TAIL_COL0, 1024)
                pltpu.async_copy(
                    tsh.at[g * 8 + j], img.at[pl.ds(off, TAIL_W)], sem_fo)

    def flat_drain(fa, fb):
        @pl.when(s <= 14)
        def _():
            _rows_drain(fa, sa8, WA1)

        @pl.when(s == 15)
        def _():
            _rows_drain(fb, sb8, WB1)
            for j in range(8):
                pltpu.make_async_copy(
                    tsh.at[j], img.at[pl.ds(0, TAIL_W)], sem_fo).wait()

    def prefetch_x(g, xv, sem):
        src = x0t.at[pl.ds(_mult(g * 8, 8), 8), pl.ds(bcol, BPT)]
        pltpu.async_copy(src, xv, sem)

    def start_gather(g, xv, sem):
        pltpu.make_async_copy(
            x0t.at[pl.ds(0, 8), pl.ds(0, BPT)], xv, sem).wait()

        for j in range(8):
            off = (g * 8 + j) * ROW_PAD

            def body(u, carry, j=j, off=off):
                ib[pl.ds(j * BPT + u * 16, 16)] = (
                    xv[j, pl.ds(u * 16, 16)] + off)
                return carry

            lax.fori_loop(0, BPT // 16, body, 0, unroll=False)

        for j in range(8):
            pltpu.async_copy(
                img.at[ib.at[pl.ds(j * BPT, BPT)]],
                gb.at[pl.ds(j * BPT, BPT)], sem_g)

    def finish_gather(g, xv, is_last_group, first):
        pltpu.make_async_copy(
            img.at[pl.ds(0, 8 * BPT)], gb, sem_g).wait()

        if is_last_group:
            def pbody(u, carry):
                gb[pl.ds(7 * BPT + u * 16, 16)] = xv[7, pl.ds(u * 16, 16)]
                return carry

            lax.fori_loop(0, BPT // 16, pbody, 0, unroll=False)

        if not first:
            drain_out()
        for j in range(8):
            pltpu.async_copy(
                gb.at[pl.ds(j * BPT, BPT)], osh.at[sa8 + j], sem_o)
        for j in range(8):
            pltpu.make_async_copy(
                gb.at[pl.ds(j * BPT, BPT)], osh.at[sa8 + j], sem_o).wait()
        pltpu.async_copy(
            osh.at[pl.ds(sa8, 8)],
            outt.at[pl.ds(_mult(g * 8, 8), 8), pl.ds(bcol, BPT)], sem_o)

    def drain_out():
        pltpu.make_async_copy(
            osh.at[pl.ds(sa8, 8)],
            outt.at[pl.ds(0, 8), pl.ds(0, BPT)], sem_o).wait()

    fshs = ((fshA0, fshB0), (fshA1, fshB1))

    xvs = (xb, xb2)
    xsems = (sem_x, sem_x2)
    flat_in(c, *fshs[0])
    prefetch_x(c, xvs[0], xsems[0])
    for i in range(12):
        g = c + 2 * i
        ph = i & 1
        flat_out(g, *fshs[ph])
        flat_drain(*fshs[ph])
        plsc.subcore_barrier()
        nxt = c + 2 * (i + 1)
        if i + 1 < 12:
            flat_in(nxt, *fshs[1 - ph])
            prefetch_x(nxt, xvs[1 - ph], xsems[1 - ph])
        else:
            @pl.when(c == 0)
            def _(nxt=nxt, ph=ph):
                flat_in(nxt, *fshs[1 - ph])
                prefetch_x(nxt, xvs[1 - ph], xsems[1 - ph])
        if i > 0:
            finish_gather(g - 2, xvs[1 - ph], is_last_group=False,
                          first=(i == 1))
        start_gather(g, xvs[ph], xsems[ph])

    @pl.when(c == 0)
    def _():
        flat_out(c + 24, *fshs[0])
        flat_drain(*fshs[0])

    plsc.subcore_barrier()

    finish_gather(c + 22, xvs[1], is_last_group=False, first=False)

    @pl.when(c == 0)
    def _():
        start_gather(c + 24, xvs[0], xsems[0])
        finish_gather(c + 24, xvs[0], is_last_group=True, first=False)

    drain_out()


@jax.jit
def _scramble(x0t, tab, tail):
    mesh = plsc.VectorSubcoreMesh(core_axis_name="c", subcore_axis_name="s")
    f = functools.partial(
        pl.kernel,
        out_type=(
            jax.ShapeDtypeStruct((POS, BATCH), jnp.int32),
            jax.ShapeDtypeStruct((IMG_SIZE,), jnp.int32),
        ),
        mesh=mesh,
        scratch_types=[
            pltpu.VMEM_SHARED((15 * 8, WA0), jnp.int32),
            pltpu.VMEM_SHARED((15 * 8, WA0), jnp.int32),
            pltpu.VMEM_SHARED((8, WB0), jnp.int32),
            pltpu.VMEM_SHARED((8, WB0), jnp.int32),
            pltpu.VMEM_SHARED((POS, TAIL_W), jnp.int32),
            pltpu.VMEM_SHARED((NS * 8, BPT), jnp.int32),
            pltpu.VMEM((8, BPT), jnp.int32),
            pltpu.VMEM((8, BPT), jnp.int32),
            pltpu.VMEM((8 * BPT,), jnp.int32),
            pltpu.VMEM((8 * BPT,), jnp.int32),
            pltpu.SemaphoreType.DMA,
            pltpu.SemaphoreType.DMA,
            pltpu.SemaphoreType.DMA,
            pltpu.SemaphoreType.DMA,
            pltpu.SemaphoreType.DMA,
            pltpu.SemaphoreType.DMA,
        ],
    )(_sc_kernel)
    return f(x0t, tab, tail)


def kernel(x0, x1, x2, scrambler):
    tail = jnp.pad(scrambler[:, TAIL_COL0:],
                   ((0, 0), (0, TAIL_W - (VOCAB - TAIL_COL0))))
    outt, _ = _scramble(x0.T, scrambler, tail)
    return (outt.T, x1, x2)


# speedup vs baseline: 4.9912x; 1.0627x over previous
import functools
import jax
import jax.numpy as jnp
from jax import lax
from jax.experimental import pallas as pl
from jax.experimental.pallas import tpu as pltpu
from jax.experimental.pallas import tpu_sc as plsc

VOCAB = 100001
POS = 200
BATCH = 16384

NC = 2
NS = 16

GROUPS = POS // 8
ROW_PAD = 100352
IMG_SIZE = POS * ROW_PAD
BPT = BATCH // NS
TAIL_COL0 = 99328
TAIL_W = 1024
WA = 6144
WA0 = 3072
WA1 = 3072
WB0 = 4096
WB1 = 3072


def _mult(x, n):
    return pl.multiple_of(x, n)


def _sc_kernel(x0t, tab, tail, outt, img,
               fshA0, fshA1, fshB0, fshB1, tsh, osh, xb, xb2, ib, gb,
               sem_fi, sem_fo, sem_x, sem_x2, sem_g, sem_o):
    c = lax.axis_index("c")
    s = lax.axis_index("s")
    col0 = _mult(s * WA, 1024)
    bcol = _mult(s * BPT, 128)
    sa8 = s * 8
    sb8 = (s - 15) * 8

    @pl.when(s == NS - 1)
    def _():
        pltpu.sync_copy(tail, tsh)

    def flat_in(g, fa, fb):
        @pl.when(s <= 14)
        def _():
            src = tab.at[pl.ds(_mult(g * 8, 8), 8), pl.ds(col0, WA0)]
            pltpu.async_copy(src, fa.at[pl.ds(sa8, 8)], sem_fi)

        @pl.when(s == 15)
        def _():
            src = tab.at[pl.ds(_mult(g * 8, 8), 8), pl.ds(col0, WB0)]
            pltpu.async_copy(src, fb.at[pl.ds(sb8, 8)], sem_fi)

    def _rows_out(g, buf, rbase, coff, w):
        for j in range(8):
            off = _mult((g * 8 + j) * ROW_PAD + coff, 1024)
            pltpu.async_copy(buf.at[rbase + j], img.at[pl.ds(off, w)],
                             sem_fo)

    def _bulk_wait(nwords, sem):
        pltpu.make_async_copy(
            img.at[pl.ds(0, nwords)], img.at[pl.ds(0, nwords)], sem).wait()

    def flat_out(g, fa, fb):
        @pl.when(s <= 14)
        def _():
            pltpu.make_async_copy(
                tab.at[pl.ds(0, 8), pl.ds(0, WA0)], fa.at[pl.ds(sa8, 8)],
                sem_fi).wait()
            _rows_out(g, fa, sa8, col0, WA0)
            _bulk_wait(8 * WA0, sem_fo)
            src = tab.at[pl.ds(_mult(g * 8, 8), 8),
                         pl.ds(col0 + WA0, WA1)]
            pltpu.async_copy(src, fa.at[pl.ds(sa8, 8), pl.ds(0, WA1)],
                             sem_fi)
            pltpu.make_async_copy(
                tab.at[pl.ds(0, 8), pl.ds(0, WA1)],
                fa.at[pl.ds(sa8, 8), pl.ds(0, WA1)], sem_fi).wait()
            for j in range(8):
                off = _mult((g * 8 + j) * ROW_PAD + col0 + WA0, 1024)
                pltpu.async_copy(
                    fa.at[sa8 + j, pl.ds(0, WA1)],
                    img.at[pl.ds(off, WA1)], sem_fo)

        @pl.when(s == 15)
        def _():
            pltpu.make_async_copy(
                tab.at[pl.ds(0, 8), pl.ds(0, WB0)], fb.at[pl.ds(sb8, 8)],
                sem_fi).wait()
            _rows_out(g, fb, sb8, col0, WB0)
            _bulk_wait(8 * WB0, sem_fo)
            src = tab.at[pl.ds(_mult(g * 8, 8), 8),
                         pl.ds(col0 + WB0, WB1)]
            pltpu.async_copy(src, fb.at[pl.ds(sb8, 8), pl.ds(0, WB1)],
                             sem_fi)
            pltpu.make_async_copy(
                tab.at[pl.ds(0, 8), pl.ds(0, WB1)],
                fb.at[pl.ds(sb8, 8), pl.ds(0, WB1)], sem_fi).wait()
            for j in range(8):
                off = _mult((g * 8 + j) * ROW_PAD + col0 + WB0, 1024)
                pltpu.async_copy(
                    fb.at[sb8 + j, pl.ds(0, WB1)],
                    img.at[pl.ds(off, WB1)], sem_fo)
            for j in range(8):
                off = _mult((g * 8 + j) * ROW_PAD + TAIL_COL0, 1024)
                pltpu.async_copy(
                    tsh.at[g * 8 + j], img.at[pl.ds(off, TAIL_W)], sem_fo)

    def flat_drain(fa, fb):
        @pl.when(s <= 14)
        def _():
            _bulk_wait(8 * WA1, sem_fo)

        @pl.when(s == 15)
        def _():
            _bulk_wait(8 * WB1 + 8 * TAIL_W, sem_fo)

    def prefetch_x(g, xv, sem):
        src = x0t.at[pl.ds(_mult(g * 8, 8), 8), pl.ds(bcol, BPT)]
        pltpu.async_copy(src, xv, sem)

    def start_gather(g, xv, sem):
        pltpu.make_async_copy(
            x0t.at[pl.ds(0, 8), pl.ds(0, BPT)], xv, sem).wait()

        for j in range(8):
            off = (g * 8 + j) * ROW_PAD

            def body(u, carry, j=j, off=off):
                ib[pl.ds(j * BPT + u * 16, 16)] = (
                    xv[j, pl.ds(u * 16, 16)] + off)
                return carry

            lax.fori_loop(0, BPT // 16, body, 0, unroll=False)

        for j in range(8):
            pltpu.async_copy(
                img.at[ib.at[pl.ds(j * BPT, BPT)]],
                gb.at[pl.ds(j * BPT, BPT)], sem_g)

    def finish_gather(g, xv, is_last_group, first):
        pltpu.make_async_copy(
            img.at[pl.ds(0, 8 * BPT)], gb, sem_g).wait()

        if is_last_group:
            def pbody(u, carry):
                gb[pl.ds(7 * BPT + u * 16, 16)] = xv[7, pl.ds(u * 16, 16)]
                return carry

            lax.fori_loop(0, BPT // 16, pbody, 0, unroll=False)

        if not first:
            drain_out()
        for j in range(8):
            pltpu.async_copy(
                gb.at[pl.ds(j * BPT, BPT)], osh.at[sa8 + j], sem_o)
        _bulk_wait(8 * BPT, sem_o)
        pltpu.async_copy(
            osh.at[pl.ds(sa8, 8)],
            outt.at[pl.ds(_mult(g * 8, 8), 8), pl.ds(bcol, BPT)], sem_o)

    def drain_out():
        pltpu.make_async_copy(
            osh.at[pl.ds(sa8, 8)],
            outt.at[pl.ds(0, 8), pl.ds(0, BPT)], sem_o).wait()

    fshs = ((fshA0, fshB0), (fshA1, fshB1))

    xvs = (xb, xb2)
    xsems = (sem_x, sem_x2)
    flat_in(c, *fshs[0])
    prefetch_x(c, xvs[0], xsems[0])
    for i in range(12):
        g = c + 2 * i
        ph = i & 1
        flat_out(g, *fshs[ph])
        flat_drain(*fshs[ph])
        plsc.subcore_barrier()
        nxt = c + 2 * (i + 1)
        if i + 1 < 12:
            flat_in(nxt, *fshs[1 - ph])
            prefetch_x(nxt, xvs[1 - ph], xsems[1 - ph])
        else:
            @pl.when(c == 0)
            def _(nxt=nxt, ph=ph):
                flat_in(nxt, *fshs[1 - ph])
                prefetch_x(nxt, xvs[1 - ph], xsems[1 - ph])
        if i > 0:
            finish_gather(g - 2, xvs[1 - ph], is_last_group=False,
                          first=(i == 1))
        start_gather(g, xvs[ph], xsems[ph])

    @pl.when(c == 0)
    def _():
        flat_out(c + 24, *fshs[0])
        flat_drain(*fshs[0])

    plsc.subcore_barrier()

    finish_gather(c + 22, xvs[1], is_last_group=False, first=False)

    @pl.when(c == 0)
    def _():
        start_gather(c + 24, xvs[0], xsems[0])
        finish_gather(c + 24, xvs[0], is_last_group=True, first=False)

    drain_out()


@jax.jit
def _scramble(x0t, tab, tail):
    mesh = plsc.VectorSubcoreMesh(core_axis_name="c", subcore_axis_name="s")
    f = functools.partial(
        pl.kernel,
        out_type=(
            jax.ShapeDtypeStruct((POS, BATCH), jnp.int32),
            jax.ShapeDtypeStruct((IMG_SIZE,), jnp.int32),
        ),
        mesh=mesh,
        scratch_types=[
            pltpu.VMEM_SHARED((15 * 8, WA0), jnp.int32),
            pltpu.VMEM_SHARED((15 * 8, WA0), jnp.int32),
            pltpu.VMEM_SHARED((8, WB0), jnp.int32),
            pltpu.VMEM_SHARED((8, WB0), jnp.int32),
            pltpu.VMEM_SHARED((POS, TAIL_W), jnp.int32),
            pltpu.VMEM_SHARED((NS * 8, BPT), jnp.int32),
            pltpu.VMEM((8, BPT), jnp.int32),
            pltpu.VMEM((8, BPT), jnp.int32),
            pltpu.VMEM((8 * BPT,), jnp.int32),
            pltpu.VMEM((8 * BPT,), jnp.int32),
            pltpu.SemaphoreType.DMA,
            pltpu.SemaphoreType.DMA,
            pltpu.SemaphoreType.DMA,
            pltpu.SemaphoreType.DMA,
            pltpu.SemaphoreType.DMA,
            pltpu.SemaphoreType.DMA,
        ],
    )(_sc_kernel)
    return f(x0t, tab, tail)


def kernel(x0, x1, x2, scrambler):
    tail = jnp.pad(scrambler[:, TAIL_COL0:],
                   ((0, 0), (0, TAIL_W - (VOCAB - TAIL_COL0))))
    outt, _ = _scramble(x0.T, scrambler, tail)
    return (outt.T, x1, x2)
